# Initial kernel scaffold; baseline (speedup 1.0000x reference)
import jax
import jax.numpy as jnp
from jax.experimental import pallas as pl


def kernel(img, flow, z):
    raise NotImplementedError("write your pallas kernel here")


# speedup vs baseline: 2.1351x; 2.1351x over previous
import functools

import jax
import jax.numpy as jnp
from jax import lax
from jax.experimental import pallas as pl
from jax.experimental.pallas import tpu as pltpu
from jax.experimental.pallas import tpu_sc as plsc

N = 4
C = 96
H = 224
W = 224
HW = H * W
NCH = 98
NGROUP = 49
NUNITS = N * NGROUP
NW = 32
LANES = 16
CHUNK = 1024
NCHUNK = HW // CHUNK
UNIT_ROUNDS = -(-NUNITS // NW)


def _prep_body(flow_ref, z_ref, idx_ref, wz_ref):
    fl = flow_ref[0]
    gx = lax.broadcasted_iota(jnp.int32, (H, W), 1).astype(jnp.float32)
    gy = lax.broadcasted_iota(jnp.int32, (H, W), 0).astype(jnp.float32)
    fx = gx + fl[0]
    fy = gy + fl[1]
    finite = jnp.isfinite(fx) & jnp.isfinite(fy)
    zero = jnp.zeros((H, W), jnp.float32)
    sfx = jnp.where(finite, fx, zero)
    sfy = jnp.where(finite, fy, zero)
    x0 = jnp.floor(sfx)
    y0 = jnp.floor(sfy)
    x1 = x0 + 1.0
    y1 = y0 + 1.0
    x0l = x0.astype(jnp.int32)
    y0l = y0.astype(jnp.int32)
    x1l = x1.astype(jnp.int32)
    y1l = y1.astype(jnp.int32)
    wx0 = x1 - sfx
    wx1 = sfx - x0
    wy0 = y1 - sfy
    wy1 = sfy - y0
    vx0 = (x0l >= 0) & (x0l < W)
    vx1 = (x1l >= 0) & (x1l < W)
    vy0 = (y0l >= 0) & (y0l < H)
    vy1 = (y1l >= 0) & (y1l < H)
    x0c = jnp.clip(x0l, 0, W - 1)
    x1c = jnp.clip(x1l, 0, W - 1)
    y0c = jnp.clip(y0l, 0, H - 1)
    y1c = jnp.clip(y1l, 0, H - 1)
    ez = jnp.exp(z_ref[0, 0])

    def emit(k, yc, xc, wgt, valid):
        idx_ref[0, k] = yc * W + xc
        wz_ref[0, k] = jnp.where(finite & valid, wgt * ez, zero)

    emit(0, y0c, x0c, wx0 * wy0, vx0 & vy0)
    emit(1, y0c, x1c, wx1 * wy0, vx1 & vy0)
    emit(2, y1c, x0c, wx0 * wy1, vx0 & vy1)
    emit(3, y1c, x1c, wx1 * wy1, vx1 & vy1)


_prep = pl.pallas_call(
    _prep_body,
    grid=(N,),
    in_specs=[
        pl.BlockSpec((1, 2, H, W), lambda b: (b, 0, 0, 0)),
        pl.BlockSpec((1, 1, H, W), lambda b: (b, 0, 0, 0)),
    ],
    out_specs=[
        pl.BlockSpec((1, 4, H, W), lambda b: (b, 0, 0, 0)),
        pl.BlockSpec((1, 4, H, W), lambda b: (b, 0, 0, 0)),
    ],
    out_shape=[
        jax.ShapeDtypeStruct((N, 4, H, W), jnp.int32),
        jax.ShapeDtypeStruct((N, 4, H, W), jnp.float32),
    ],
)


def _splat_body(idx_hbm, wz_hbm, img_hbm, out_hbm, acc, ibuf, wbuf, fbuf, sem):
    wid = lax.axis_index("s") * 2 + lax.axis_index("c")

    def unit_body(ui, _):
        u = wid + ui * NW

        @pl.when(u < NUNITS)
        def _():
            b = u // NGROUP
            cg = u % NGROUP
            c0 = 2 * cg
            is_norm = cg == (NGROUP - 1)

            def zero_body(i, _):
                base = i * (32 * LANES)
                for j in range(32):
                    acc[pl.ds(base + j * LANES, LANES)] = jnp.zeros(
                        (LANES,), jnp.float32)
                return _

            lax.fori_loop(0, (2 * HW) // (32 * LANES), zero_body, None)

            def chunk_body(ci, _):
                p0 = ci * CHUNK
                copies = []
                for k in range(4):
                    copies.append(pltpu.async_copy(
                        idx_hbm.at[b, k, pl.ds(p0, CHUNK)],
                        ibuf.at[pl.ds(k * CHUNK, CHUNK)], sem))
                    copies.append(pltpu.async_copy(
                        wz_hbm.at[b, k, pl.ds(p0, CHUNK)],
                        wbuf.at[pl.ds(k * CHUNK, CHUNK)], sem))
                for cp in copies:
                    cp.wait()

                @pl.when(jnp.logical_not(is_norm))
                def _():
                    fc = []
                    for j in range(2):
                        fc.append(pltpu.async_copy(
                            img_hbm.at[b, c0 + j, pl.ds(p0, CHUNK)],
                            fbuf.at[pl.ds(j * CHUNK, CHUNK)], sem))
                    for cp in fc:
                        cp.wait()

                    def step(s, _):
                        o = s * LANES
                        f0 = fbuf[pl.ds(o, LANES)]
                        f1 = fbuf[pl.ds(CHUNK + o, LANES)]
                        for k in range(4):
                            iv = ibuf[pl.ds(k * CHUNK + o, LANES)]
                            wv = wbuf[pl.ds(k * CHUNK + o, LANES)]
                            plsc.addupdate_scatter(acc, [iv], wv * f0)
                            plsc.addupdate_scatter(acc, [iv + HW], wv * f1)
                        return _

                    lax.fori_loop(0, CHUNK // LANES, step, None)

                @pl.when(is_norm)
                def _():
                    def step(s, _):
                        o = s * LANES
                        for k in range(4):
                            iv = ibuf[pl.ds(k * CHUNK + o, LANES)]
                            wv = wbuf[pl.ds(k * CHUNK + o, LANES)]
                            plsc.addupdate_scatter(acc, [iv], wv)
                        return _

                    lax.fori_loop(0, CHUNK // LANES, step, None)

                return _

            lax.fori_loop(0, NCHUNK, chunk_body, None)
            pltpu.sync_copy(acc.at[pl.ds(0, HW)], out_hbm.at[b, c0])
            pltpu.sync_copy(acc.at[pl.ds(HW, HW)], out_hbm.at[b, c0 + 1])

        return _

    lax.fori_loop(0, UNIT_ROUNDS, unit_body, None)


@functools.cache
def _get_splat():
    return pl.kernel(
        _splat_body,
        out_type=jax.ShapeDtypeStruct((N, NCH, HW), jnp.float32),
        mesh=plsc.VectorSubcoreMesh(
            core_axis_name="c", subcore_axis_name="s",
            num_cores=2, num_subcores=16),
        scratch_types=[
            pltpu.VMEM((2 * HW,), jnp.float32),
            pltpu.VMEM((4 * CHUNK,), jnp.int32),
            pltpu.VMEM((4 * CHUNK,), jnp.float32),
            pltpu.VMEM((2 * CHUNK,), jnp.float32),
            pltpu.SemaphoreType.DMA,
        ],
        compiler_params=pltpu.CompilerParams(needs_layout_passes=False),
    )


CB = 16


def _norm_body(acc_ref, nrm_ref, o_ref):
    nv = nrm_ref[0, 0]
    d = jnp.where(nv == 0.0, jnp.ones_like(nv), nv)
    o_ref[0] = acc_ref[0] / d[None, :, :]


_norm = pl.pallas_call(
    _norm_body,
    grid=(N, C // CB),
    in_specs=[
        pl.BlockSpec((1, CB, H, W), lambda b, c: (b, c, 0, 0)),
        pl.BlockSpec((1, 1, H, W), lambda b, c: (b, 0, 0, 0)),
    ],
    out_specs=pl.BlockSpec((1, CB, H, W), lambda b, c: (b, c, 0, 0)),
    out_shape=jax.ShapeDtypeStruct((N, C, H, W), jnp.float32),
)


@jax.jit
def kernel(img, flow, z):
    idx, wz = _prep(flow, z)
    buf = _get_splat()(idx.reshape(N, 4, HW), wz.reshape(N, 4, HW),
                       img.reshape(N, C, HW))
    buf = buf.reshape(N, NCH, H, W)
    return _norm(buf[:, :C], buf[:, C:C + 1])


# speedup vs baseline: 3.1438x; 1.4725x over previous
import functools

import jax
import jax.numpy as jnp
from jax import lax
from jax.experimental import pallas as pl
from jax.experimental.pallas import tpu as pltpu
from jax.experimental.pallas import tpu_sc as plsc

N = 4
C = 96
H = 224
W = 224
HW = H * W
NCH = 98
NGROUP = 49
NUNITS = N * NGROUP
NW = 32
LANES = 16
CHUNK = 784
NCHUNK = HW // CHUNK
UNIT_ROUNDS = -(-NUNITS // NW)


def _prep_body(flow_ref, z_ref, idx_ref, wz_ref):
    fl = flow_ref[0]
    gx = lax.broadcasted_iota(jnp.int32, (H, W), 1).astype(jnp.float32)
    gy = lax.broadcasted_iota(jnp.int32, (H, W), 0).astype(jnp.float32)
    fx = gx + fl[0]
    fy = gy + fl[1]
    finite = jnp.isfinite(fx) & jnp.isfinite(fy)
    zero = jnp.zeros((H, W), jnp.float32)
    sfx = jnp.where(finite, fx, zero)
    sfy = jnp.where(finite, fy, zero)
    x0 = jnp.floor(sfx)
    y0 = jnp.floor(sfy)
    x1 = x0 + 1.0
    y1 = y0 + 1.0
    x0l = x0.astype(jnp.int32)
    y0l = y0.astype(jnp.int32)
    x1l = x1.astype(jnp.int32)
    y1l = y1.astype(jnp.int32)
    wx0 = x1 - sfx
    wx1 = sfx - x0
    wy0 = y1 - sfy
    wy1 = sfy - y0
    vx0 = (x0l >= 0) & (x0l < W)
    vx1 = (x1l >= 0) & (x1l < W)
    vy0 = (y0l >= 0) & (y0l < H)
    vy1 = (y1l >= 0) & (y1l < H)
    x0c = jnp.clip(x0l, 0, W - 1)
    x1c = jnp.clip(x1l, 0, W - 1)
    y0c = jnp.clip(y0l, 0, H - 1)
    y1c = jnp.clip(y1l, 0, H - 1)
    ez = jnp.exp(z_ref[0, 0])

    def emit(k, yc, xc, wgt, valid):
        idx_ref[0, k] = yc * W + xc
        wz_ref[0, k] = jnp.where(finite & valid, wgt * ez, zero)

    emit(0, y0c, x0c, wx0 * wy0, vx0 & vy0)
    emit(1, y0c, x1c, wx1 * wy0, vx1 & vy0)
    emit(2, y1c, x0c, wx0 * wy1, vx0 & vy1)
    emit(3, y1c, x1c, wx1 * wy1, vx1 & vy1)


_prep = pl.pallas_call(
    _prep_body,
    grid=(N,),
    in_specs=[
        pl.BlockSpec((1, 2, H, W), lambda b: (b, 0, 0, 0)),
        pl.BlockSpec((1, 1, H, W), lambda b: (b, 0, 0, 0)),
    ],
    out_specs=[
        pl.BlockSpec((1, 4, H, W), lambda b: (b, 0, 0, 0)),
        pl.BlockSpec((1, 4, H, W), lambda b: (b, 0, 0, 0)),
    ],
    out_shape=[
        jax.ShapeDtypeStruct((N, 4, H, W), jnp.int32),
        jax.ShapeDtypeStruct((N, 4, H, W), jnp.float32),
    ],
)


def _splat_body(idx_hbm, wz_hbm, img_hbm, out_hbm, acc,
                ib0, wb0, fb0, ib1, wb1, fb1, sem0, sem1):
    wid = lax.axis_index("s") * 2 + lax.axis_index("c")

    def unit_body(ui, _):
        u = wid + ui * NW

        @pl.when(u < NUNITS)
        def _():
            b = u // NGROUP
            cg = u % NGROUP
            c0 = 2 * cg
            is_norm = cg == (NGROUP - 1)
            c0f = jnp.minimum(c0, C - 2)

            def fire(ci, ib, wb, fb, sem):
                pltpu.async_copy(idx_hbm.at[b, ci], ib, sem)
                pltpu.async_copy(wz_hbm.at[b, ci], wb, sem)
                p0 = ci * CHUNK
                pltpu.async_copy(
                    img_hbm.at[b, c0f, pl.ds(p0, CHUNK)], fb.at[0], sem)
                pltpu.async_copy(
                    img_hbm.at[b, c0f + 1, pl.ds(p0, CHUNK)], fb.at[1], sem)

            def wait_slot(ib, wb, fb, sem):
                pltpu.make_async_copy(idx_hbm.at[0, 0], ib, sem).wait()
                pltpu.make_async_copy(wz_hbm.at[0, 0], wb, sem).wait()
                pltpu.make_async_copy(
                    img_hbm.at[0, 0, pl.ds(0, CHUNK)], fb.at[0], sem).wait()
                pltpu.make_async_copy(
                    img_hbm.at[0, 0, pl.ds(0, CHUNK)], fb.at[1], sem).wait()

            def compute(ib, wb, fb):
                @pl.when(jnp.logical_not(is_norm))
                def _():
                    def step(s, _):
                        o = s * LANES
                        f0 = fb[0, pl.ds(o, LANES)]
                        f1 = fb[1, pl.ds(o, LANES)]
                        for k in range(4):
                            iv = ib[k, pl.ds(o, LANES)]
                            wv = wb[k, pl.ds(o, LANES)]
                            plsc.addupdate_scatter(acc, [iv], wv * f0)
                            plsc.addupdate_scatter(acc, [iv + HW], wv * f1)
                        return _

                    lax.fori_loop(0, CHUNK // LANES, step, None)

                @pl.when(is_norm)
                def _():
                    def step(s, _):
                        o = s * LANES
                        for k in range(4):
                            iv = ib[k, pl.ds(o, LANES)]
                            wv = wb[k, pl.ds(o, LANES)]
                            plsc.addupdate_scatter(acc, [iv], wv)
                        return _

                    lax.fori_loop(0, CHUNK // LANES, step, None)

            fire(0, ib0, wb0, fb0, sem0)

            def zero_body(i, _):
                base = i * (32 * LANES)
                for j in range(32):
                    acc[pl.ds(base + j * LANES, LANES)] = jnp.zeros(
                        (LANES,), jnp.float32)
                return _

            lax.fori_loop(0, (2 * HW) // (32 * LANES), zero_body, None)

            def pair_body(j, _):
                ci = 2 * j
                fire(ci + 1, ib1, wb1, fb1, sem1)
                wait_slot(ib0, wb0, fb0, sem0)
                compute(ib0, wb0, fb0)

                @pl.when(ci + 2 < NCHUNK)
                def _():
                    fire(ci + 2, ib0, wb0, fb0, sem0)

                wait_slot(ib1, wb1, fb1, sem1)
                compute(ib1, wb1, fb1)
                return _

            lax.fori_loop(0, NCHUNK // 2, pair_body, None)
            pltpu.sync_copy(acc.at[pl.ds(0, HW)], out_hbm.at[b, c0])
            pltpu.sync_copy(acc.at[pl.ds(HW, HW)], out_hbm.at[b, c0 + 1])

        return _

    lax.fori_loop(0, UNIT_ROUNDS, unit_body, None)


@functools.cache
def _get_splat():
    return pl.kernel(
        _splat_body,
        out_type=jax.ShapeDtypeStruct((N, NCH, HW), jnp.float32),
        mesh=plsc.VectorSubcoreMesh(
            core_axis_name="c", subcore_axis_name="s",
            num_cores=2, num_subcores=16),
        scratch_types=[
            pltpu.VMEM((2 * HW,), jnp.float32),
            pltpu.VMEM((4, CHUNK), jnp.int32),
            pltpu.VMEM((4, CHUNK), jnp.float32),
            pltpu.VMEM((2, CHUNK), jnp.float32),
            pltpu.VMEM((4, CHUNK), jnp.int32),
            pltpu.VMEM((4, CHUNK), jnp.float32),
            pltpu.VMEM((2, CHUNK), jnp.float32),
            pltpu.SemaphoreType.DMA,
            pltpu.SemaphoreType.DMA,
        ],
        compiler_params=pltpu.CompilerParams(
            needs_layout_passes=False, use_tc_tiling_on_sc=False),
    )


CB = 16


def _norm_body(acc_ref, nrm_ref, o_ref):
    nv = nrm_ref[0, 0]
    d = jnp.where(nv == 0.0, jnp.ones_like(nv), nv)
    o_ref[0] = acc_ref[0] / d[None, :, :]


_norm = pl.pallas_call(
    _norm_body,
    grid=(N, C // CB),
    in_specs=[
        pl.BlockSpec((1, CB, H, W), lambda b, c: (b, c, 0, 0)),
        pl.BlockSpec((1, 1, H, W), lambda b, c: (b, 0, 0, 0)),
    ],
    out_specs=pl.BlockSpec((1, CB, H, W), lambda b, c: (b, c, 0, 0)),
    out_shape=jax.ShapeDtypeStruct((N, C, H, W), jnp.float32),
)


@jax.jit
def kernel(img, flow, z):
    idx, wz = _prep(flow, z)
    idxc = idx.reshape(N, 4, NCHUNK, CHUNK).transpose(0, 2, 1, 3)
    wzc = wz.reshape(N, 4, NCHUNK, CHUNK).transpose(0, 2, 1, 3)
    buf = _get_splat()(idxc, wzc, img.reshape(N, C, HW))
    buf = buf.reshape(N, NCH, H, W)
    return _norm(buf[:, :C], buf[:, C:C + 1])


# speedup vs baseline: 3.2053x; 1.0196x over previous
import functools

import jax
import jax.numpy as jnp
from jax import lax
from jax.experimental import pallas as pl
from jax.experimental.pallas import tpu as pltpu
from jax.experimental.pallas import tpu_sc as plsc

N = 4
C = 96
H = 224
W = 224
HW = H * W
NCH = 98
NGROUP = 49
NUNITS = N * NGROUP
NW = 32
LANES = 16
CHUNK = 896
NCHUNK = HW // CHUNK
NPAIR = NCHUNK // 2
NGRP = 48
REG_ROUNDS = (N * NGRP) // NW


def _prep_body(flow_ref, z_ref, idx_ref, wz_ref):
    fl = flow_ref[0]
    gx = lax.broadcasted_iota(jnp.int32, (H, W), 1).astype(jnp.float32)
    gy = lax.broadcasted_iota(jnp.int32, (H, W), 0).astype(jnp.float32)
    fx = gx + fl[0]
    fy = gy + fl[1]
    finite = jnp.isfinite(fx) & jnp.isfinite(fy)
    zero = jnp.zeros((H, W), jnp.float32)
    sfx = jnp.where(finite, fx, zero)
    sfy = jnp.where(finite, fy, zero)
    x0 = jnp.floor(sfx)
    y0 = jnp.floor(sfy)
    x1 = x0 + 1.0
    y1 = y0 + 1.0
    x0l = x0.astype(jnp.int32)
    y0l = y0.astype(jnp.int32)
    x1l = x1.astype(jnp.int32)
    y1l = y1.astype(jnp.int32)
    wx0 = x1 - sfx
    wx1 = sfx - x0
    wy0 = y1 - sfy
    wy1 = sfy - y0
    vx0 = (x0l >= 0) & (x0l < W)
    vx1 = (x1l >= 0) & (x1l < W)
    vy0 = (y0l >= 0) & (y0l < H)
    vy1 = (y1l >= 0) & (y1l < H)
    x0c = jnp.clip(x0l, 0, W - 1)
    x1c = jnp.clip(x1l, 0, W - 1)
    y0c = jnp.clip(y0l, 0, H - 1)
    y1c = jnp.clip(y1l, 0, H - 1)
    ez = jnp.exp(z_ref[0, 0])

    def emit(k, yc, xc, wgt, valid):
        idx_ref[0, k] = yc * W + xc
        wz_ref[0, k] = jnp.where(finite & valid, wgt * ez, zero)

    emit(0, y0c, x0c, wx0 * wy0, vx0 & vy0)
    emit(1, y0c, x1c, wx1 * wy0, vx1 & vy0)
    emit(2, y1c, x0c, wx0 * wy1, vx0 & vy1)
    emit(3, y1c, x1c, wx1 * wy1, vx1 & vy1)


_prep = pl.pallas_call(
    _prep_body,
    grid=(N,),
    in_specs=[
        pl.BlockSpec((1, 2, H, W), lambda b: (b, 0, 0, 0)),
        pl.BlockSpec((1, 1, H, W), lambda b: (b, 0, 0, 0)),
    ],
    out_specs=[
        pl.BlockSpec((1, 4, H, W), lambda b: (b, 0, 0, 0)),
        pl.BlockSpec((1, 4, H, W), lambda b: (b, 0, 0, 0)),
    ],
    out_shape=[
        jax.ShapeDtypeStruct((N, 4, H, W), jnp.int32),
        jax.ShapeDtypeStruct((N, 4, H, W), jnp.float32),
    ],
)


def _splat_body(idx_hbm, wz_hbm, img_hbm, out_hbm, acc,
                ib0, wb0, fb0, ib1, wb1, fb1, sem0, sem1):
    wid = lax.axis_index("s") * 2 + lax.axis_index("c")

    def zero_acc():
        def zero_body(i, _):
            base = i * (32 * LANES)
            for j in range(32):
                acc[pl.ds(base + j * LANES, LANES)] = jnp.zeros(
                    (LANES,), jnp.float32)
            return _

        lax.fori_loop(0, (2 * HW) // (32 * LANES), zero_body, None)

    def process(b, c0, is_norm):
        def fire(ci, ib, wb, fb, sem):
            pltpu.async_copy(idx_hbm.at[b, ci], ib, sem)
            pltpu.async_copy(wz_hbm.at[b, ci], wb, sem)
            if not is_norm:
                p0 = ci * CHUNK
                pltpu.async_copy(
                    img_hbm.at[b, c0, pl.ds(p0, CHUNK)], fb.at[0], sem)
                pltpu.async_copy(
                    img_hbm.at[b, c0 + 1, pl.ds(p0, CHUNK)], fb.at[1], sem)

        def wait_slot(ib, wb, fb, sem):
            pltpu.make_async_copy(idx_hbm.at[0, 0], ib, sem).wait()
            pltpu.make_async_copy(wz_hbm.at[0, 0], wb, sem).wait()
            if not is_norm:
                pltpu.make_async_copy(
                    img_hbm.at[0, 0, pl.ds(0, CHUNK)], fb.at[0], sem).wait()
                pltpu.make_async_copy(
                    img_hbm.at[0, 0, pl.ds(0, CHUNK)], fb.at[1], sem).wait()

        def compute(ib, wb, fb):
            def substep(o):
                if is_norm:
                    for k in range(4):
                        iv = ib[k, pl.ds(o, LANES)]
                        wv = wb[k, pl.ds(o, LANES)]
                        plsc.addupdate_scatter(acc, [iv], wv)
                else:
                    f0 = fb[0, pl.ds(o, LANES)]
                    f1 = fb[1, pl.ds(o, LANES)]
                    for k in range(4):
                        iv = ib[k, pl.ds(o, LANES)]
                        wv = wb[k, pl.ds(o, LANES)]
                        plsc.addupdate_scatter(acc, [iv], wv * f0)
                        plsc.addupdate_scatter(acc, [iv + HW], wv * f1)

            def step(s, _):
                o = s * (2 * LANES)
                substep(o)
                substep(o + LANES)
                return _

            lax.fori_loop(0, CHUNK // (2 * LANES), step, None)

        fire(0, ib0, wb0, fb0, sem0)
        zero_acc()

        def pair_body(j, _):
            ci = 2 * j
            fire(ci + 1, ib1, wb1, fb1, sem1)
            wait_slot(ib0, wb0, fb0, sem0)
            compute(ib0, wb0, fb0)

            @pl.when(ci + 2 < NCHUNK)
            def _():
                fire(ci + 2, ib0, wb0, fb0, sem0)

            wait_slot(ib1, wb1, fb1, sem1)
            compute(ib1, wb1, fb1)
            return _

        lax.fori_loop(0, NPAIR, pair_body, None)
        pltpu.sync_copy(acc.at[pl.ds(0, HW)], out_hbm.at[b, c0])
        pltpu.sync_copy(acc.at[pl.ds(HW, HW)], out_hbm.at[b, c0 + 1])

    def reg_body(ui, _):
        rid = wid + ui * NW
        process(rid // NGRP, 2 * (rid % NGRP), False)
        return _

    lax.fori_loop(0, REG_ROUNDS, reg_body, None)

    @pl.when(wid >= NW - N)
    def _():
        process(wid - (NW - N), C, True)


@functools.cache
def _get_splat():
    return pl.kernel(
        _splat_body,
        out_type=jax.ShapeDtypeStruct((N, NCH, HW), jnp.float32),
        mesh=plsc.VectorSubcoreMesh(
            core_axis_name="c", subcore_axis_name="s",
            num_cores=2, num_subcores=16),
        scratch_types=[
            pltpu.VMEM((2 * HW,), jnp.float32),
            pltpu.VMEM((4, CHUNK), jnp.int32),
            pltpu.VMEM((4, CHUNK), jnp.float32),
            pltpu.VMEM((2, CHUNK), jnp.float32),
            pltpu.VMEM((4, CHUNK), jnp.int32),
            pltpu.VMEM((4, CHUNK), jnp.float32),
            pltpu.VMEM((2, CHUNK), jnp.float32),
            pltpu.SemaphoreType.DMA,
            pltpu.SemaphoreType.DMA,
        ],
        compiler_params=pltpu.CompilerParams(
            needs_layout_passes=False, use_tc_tiling_on_sc=False),
    )


CB = 16


def _norm_body(acc_ref, nrm_ref, o_ref):
    nv = nrm_ref[0, 0]
    d = jnp.where(nv == 0.0, jnp.ones_like(nv), nv)
    o_ref[0] = acc_ref[0] / d[None, :, :]


_norm = pl.pallas_call(
    _norm_body,
    grid=(N, C // CB),
    in_specs=[
        pl.BlockSpec((1, CB, H, W), lambda b, c: (b, c, 0, 0)),
        pl.BlockSpec((1, 1, H, W), lambda b, c: (b, 0, 0, 0)),
    ],
    out_specs=pl.BlockSpec((1, CB, H, W), lambda b, c: (b, c, 0, 0)),
    out_shape=jax.ShapeDtypeStruct((N, C, H, W), jnp.float32),
)


@jax.jit
def kernel(img, flow, z):
    idx, wz = _prep(flow, z)
    idxc = idx.reshape(N, 4, NCHUNK, CHUNK).transpose(0, 2, 1, 3)
    wzc = wz.reshape(N, 4, NCHUNK, CHUNK).transpose(0, 2, 1, 3)
    buf = _get_splat()(idxc, wzc, img.reshape(N, C, HW))
    buf = buf.reshape(N, NCH, H, W)
    return _norm(buf[:, :C], buf[:, C:C + 1])


# speedup vs baseline: 3.2165x; 1.0035x over previous
import functools

import jax
import jax.numpy as jnp
from jax import lax
from jax.experimental import pallas as pl
from jax.experimental.pallas import tpu as pltpu
from jax.experimental.pallas import tpu_sc as plsc

N = 4
C = 96
H = 224
W = 224
HW = H * W
NCH = 98
NGROUP = 49
NUNITS = N * NGROUP
NW = 32
LANES = 16
CHUNK = 896
NCHUNK = HW // CHUNK
NPAIR = NCHUNK // 2
NGRP = 48
REG_ROUNDS = (N * NGRP) // NW


def _prep_body(flow_ref, z_ref, idx_ref, wz_ref):
    fl = flow_ref[0]
    gx = lax.broadcasted_iota(jnp.int32, (H, W), 1).astype(jnp.float32)
    gy = lax.broadcasted_iota(jnp.int32, (H, W), 0).astype(jnp.float32)
    fx = gx + fl[0]
    fy = gy + fl[1]
    finite = jnp.isfinite(fx) & jnp.isfinite(fy)
    zero = jnp.zeros((H, W), jnp.float32)
    sfx = jnp.where(finite, fx, zero)
    sfy = jnp.where(finite, fy, zero)
    x0 = jnp.floor(sfx)
    y0 = jnp.floor(sfy)
    x1 = x0 + 1.0
    y1 = y0 + 1.0
    x0l = x0.astype(jnp.int32)
    y0l = y0.astype(jnp.int32)
    x1l = x1.astype(jnp.int32)
    y1l = y1.astype(jnp.int32)
    wx0 = x1 - sfx
    wx1 = sfx - x0
    wy0 = y1 - sfy
    wy1 = sfy - y0
    vx0 = (x0l >= 0) & (x0l < W)
    vx1 = (x1l >= 0) & (x1l < W)
    vy0 = (y0l >= 0) & (y0l < H)
    vy1 = (y1l >= 0) & (y1l < H)
    x0c = jnp.clip(x0l, 0, W - 1)
    x1c = jnp.clip(x1l, 0, W - 1)
    y0c = jnp.clip(y0l, 0, H - 1)
    y1c = jnp.clip(y1l, 0, H - 1)
    ez = jnp.exp(z_ref[0, 0])

    def emit(k, yc, xc, wgt, valid):
        idx_ref[0, k] = yc * W + xc
        wz_ref[0, k] = jnp.where(finite & valid, wgt * ez, zero)

    emit(0, y0c, x0c, wx0 * wy0, vx0 & vy0)
    emit(1, y0c, x1c, wx1 * wy0, vx1 & vy0)
    emit(2, y1c, x0c, wx0 * wy1, vx0 & vy1)
    emit(3, y1c, x1c, wx1 * wy1, vx1 & vy1)


_prep = pl.pallas_call(
    _prep_body,
    grid=(N,),
    in_specs=[
        pl.BlockSpec((1, 2, H, W), lambda b: (b, 0, 0, 0)),
        pl.BlockSpec((1, 1, H, W), lambda b: (b, 0, 0, 0)),
    ],
    out_specs=[
        pl.BlockSpec((1, 4, H, W), lambda b: (b, 0, 0, 0)),
        pl.BlockSpec((1, 4, H, W), lambda b: (b, 0, 0, 0)),
    ],
    out_shape=[
        jax.ShapeDtypeStruct((N, 4, H, W), jnp.int32),
        jax.ShapeDtypeStruct((N, 4, H, W), jnp.float32),
    ],
)


def _splat_body(idx_hbm, wz_hbm, img_hbm, out_hbm, acc,
                ib0, wb0, fb0, ib1, wb1, fb1, sem0, sem1):
    wid = lax.axis_index("s") * 2 + lax.axis_index("c")

    def zero_acc():
        def zero_body(i, _):
            base = i * (32 * LANES)
            for j in range(32):
                acc[pl.ds(base + j * LANES, LANES)] = jnp.zeros(
                    (LANES,), jnp.float32)
            return _

        lax.fori_loop(0, (2 * HW) // (32 * LANES), zero_body, None)

    def process(b, c0, is_norm):
        def fire(ci, ib, wb, fb, sem):
            pltpu.async_copy(idx_hbm.at[b, ci], ib, sem)
            pltpu.async_copy(wz_hbm.at[b, ci], wb, sem)
            if not is_norm:
                p0 = ci * CHUNK
                pltpu.async_copy(
                    img_hbm.at[b, c0, pl.ds(p0, CHUNK)], fb.at[0], sem)
                pltpu.async_copy(
                    img_hbm.at[b, c0 + 1, pl.ds(p0, CHUNK)], fb.at[1], sem)

        def wait_slot(ib, wb, fb, sem):
            pltpu.make_async_copy(idx_hbm.at[0, 0], ib, sem).wait()
            pltpu.make_async_copy(wz_hbm.at[0, 0], wb, sem).wait()
            if not is_norm:
                pltpu.make_async_copy(
                    img_hbm.at[0, 0, pl.ds(0, CHUNK)], fb.at[0], sem).wait()
                pltpu.make_async_copy(
                    img_hbm.at[0, 0, pl.ds(0, CHUNK)], fb.at[1], sem).wait()

        def compute(ib, wb, fb):
            def substep(o):
                if is_norm:
                    for k in range(4):
                        iv = ib[k, pl.ds(o, LANES)]
                        wv = wb[k, pl.ds(o, LANES)]
                        plsc.addupdate_scatter(acc, [iv], wv)
                else:
                    f0 = fb[0, pl.ds(o, LANES)]
                    f1 = fb[1, pl.ds(o, LANES)]
                    for k in range(4):
                        iv = ib[k, pl.ds(o, LANES)]
                        wv = wb[k, pl.ds(o, LANES)]
                        plsc.addupdate_scatter(acc, [iv], wv * f0)
                        plsc.addupdate_scatter(acc, [iv + HW], wv * f1)

            def step(s, _):
                o = s * (2 * LANES)
                substep(o)
                substep(o + LANES)
                return _

            lax.fori_loop(0, CHUNK // (2 * LANES), step, None)

        fire(0, ib0, wb0, fb0, sem0)
        zero_acc()

        def pair_body(j, _):
            ci = 2 * j
            fire(ci + 1, ib1, wb1, fb1, sem1)
            wait_slot(ib0, wb0, fb0, sem0)
            compute(ib0, wb0, fb0)

            @pl.when(ci + 2 < NCHUNK)
            def _():
                fire(ci + 2, ib0, wb0, fb0, sem0)

            wait_slot(ib1, wb1, fb1, sem1)
            compute(ib1, wb1, fb1)
            return _

        lax.fori_loop(0, NPAIR, pair_body, None)
        pltpu.sync_copy(acc.at[pl.ds(0, HW)], out_hbm.at[b, c0])
        pltpu.sync_copy(acc.at[pl.ds(HW, HW)], out_hbm.at[b, c0 + 1])

    def reg_body(ui, _):
        rid = wid + ui * NW
        process(rid // NGRP, 2 * (rid % NGRP), False)
        return _

    lax.fori_loop(0, REG_ROUNDS, reg_body, None)

    @pl.when(wid >= NW - N)
    def _():
        process(wid - (NW - N), C, True)


@functools.cache
def _get_splat():
    return pl.kernel(
        _splat_body,
        out_type=jax.ShapeDtypeStruct((N, NCH, HW), jnp.float32),
        mesh=plsc.VectorSubcoreMesh(
            core_axis_name="c", subcore_axis_name="s",
            num_cores=2, num_subcores=16),
        scratch_types=[
            pltpu.VMEM((2 * HW,), jnp.float32),
            pltpu.VMEM((4, CHUNK), jnp.int32),
            pltpu.VMEM((4, CHUNK), jnp.float32),
            pltpu.VMEM((2, CHUNK), jnp.float32),
            pltpu.VMEM((4, CHUNK), jnp.int32),
            pltpu.VMEM((4, CHUNK), jnp.float32),
            pltpu.VMEM((2, CHUNK), jnp.float32),
            pltpu.SemaphoreType.DMA,
            pltpu.SemaphoreType.DMA,
        ],
        compiler_params=pltpu.CompilerParams(needs_layout_passes=False),
    )


CB = 16


def _norm_body(acc_ref, nrm_ref, o_ref):
    nv = nrm_ref[0, 0]
    d = jnp.where(nv == 0.0, jnp.ones_like(nv), nv)
    o_ref[0] = acc_ref[0] / d[None, :, :]


_norm = pl.pallas_call(
    _norm_body,
    grid=(N, C // CB),
    in_specs=[
        pl.BlockSpec((1, CB, H, W), lambda b, c: (b, c, 0, 0)),
        pl.BlockSpec((1, 1, H, W), lambda b, c: (b, 0, 0, 0)),
    ],
    out_specs=pl.BlockSpec((1, CB, H, W), lambda b, c: (b, c, 0, 0)),
    out_shape=jax.ShapeDtypeStruct((N, C, H, W), jnp.float32),
)


@jax.jit
def kernel(img, flow, z):
    idx, wz = _prep(flow, z)
    idxc = idx.reshape(N, 4, NCHUNK, CHUNK).transpose(0, 2, 1, 3)
    wzc = wz.reshape(N, 4, NCHUNK, CHUNK).transpose(0, 2, 1, 3)
    buf = _get_splat()(idxc, wzc, img.reshape(N, C, HW))
    buf = buf.reshape(N, NCH, H, W)
    return _norm(buf[:, :C], buf[:, C:C + 1])


# speedup vs baseline: 3.7740x; 1.1733x over previous
import functools

import jax
import jax.numpy as jnp
from jax import lax
from jax.experimental import pallas as pl
from jax.experimental.pallas import tpu as pltpu
from jax.experimental.pallas import tpu_sc as plsc

N = 4
C = 96
H = 224
W = 224
HW = H * W
NCH = 98
NGROUP = 49
NUNITS = N * NGROUP
NW = 32
LANES = 16
CHUNK = 896
NCHUNK = HW // CHUNK
NPAIR = NCHUNK // 2
NGRP = 48
REG_ROUNDS = (N * NGRP) // NW


def _prep_body(flow_ref, z_ref, idx_ref, wz_ref):
    fl = flow_ref[0]
    gx = lax.broadcasted_iota(jnp.int32, (H, W), 1).astype(jnp.float32)
    gy = lax.broadcasted_iota(jnp.int32, (H, W), 0).astype(jnp.float32)
    fx = gx + fl[0]
    fy = gy + fl[1]
    finite = jnp.isfinite(fx) & jnp.isfinite(fy)
    zero = jnp.zeros((H, W), jnp.float32)
    sfx = jnp.where(finite, fx, zero)
    sfy = jnp.where(finite, fy, zero)
    x0 = jnp.floor(sfx)
    y0 = jnp.floor(sfy)
    x1 = x0 + 1.0
    y1 = y0 + 1.0
    x0l = x0.astype(jnp.int32)
    y0l = y0.astype(jnp.int32)
    x1l = x1.astype(jnp.int32)
    y1l = y1.astype(jnp.int32)
    wx0 = x1 - sfx
    wx1 = sfx - x0
    wy0 = y1 - sfy
    wy1 = sfy - y0
    vx0 = (x0l >= 0) & (x0l < W)
    vx1 = (x1l >= 0) & (x1l < W)
    vy0 = (y0l >= 0) & (y0l < H)
    vy1 = (y1l >= 0) & (y1l < H)
    x0c = jnp.clip(x0l, 0, W - 1)
    x1c = jnp.clip(x1l, 0, W - 1)
    y0c = jnp.clip(y0l, 0, H - 1)
    y1c = jnp.clip(y1l, 0, H - 1)
    ez = jnp.exp(z_ref[0, 0])

    def emit(k, yc, xc, wgt, valid):
        idx_ref[0, k] = yc * W + xc
        wz_ref[0, k] = jnp.where(finite & valid, wgt * ez, zero)

    emit(0, y0c, x0c, wx0 * wy0, vx0 & vy0)
    emit(1, y0c, x1c, wx1 * wy0, vx1 & vy0)
    emit(2, y1c, x0c, wx0 * wy1, vx0 & vy1)
    emit(3, y1c, x1c, wx1 * wy1, vx1 & vy1)


_prep = pl.pallas_call(
    _prep_body,
    grid=(N,),
    in_specs=[
        pl.BlockSpec((1, 2, H, W), lambda b: (b, 0, 0, 0)),
        pl.BlockSpec((1, 1, H, W), lambda b: (b, 0, 0, 0)),
    ],
    out_specs=[
        pl.BlockSpec((1, 4, H, W), lambda b: (b, 0, 0, 0)),
        pl.BlockSpec((1, 4, H, W), lambda b: (b, 0, 0, 0)),
    ],
    out_shape=[
        jax.ShapeDtypeStruct((N, 4, H, W), jnp.int32),
        jax.ShapeDtypeStruct((N, 4, H, W), jnp.float32),
    ],
)


def _splat_body(idx_hbm, wz_hbm, img_hbm, out_hbm, acc,
                ib0, wb0, fb0, ib1, wb1, fb1, sem0, sem1):
    wid = lax.axis_index("s") * 2 + lax.axis_index("c")

    def zero_acc():
        def zero_body(i, _):
            base = i * (32 * LANES)
            for j in range(32):
                acc[pl.ds(base + j * LANES, LANES)] = jnp.zeros(
                    (LANES,), jnp.float32)
            return _

        lax.fori_loop(0, (2 * HW) // (32 * LANES), zero_body, None)

    def process(b, c0, is_norm):
        def fire(ci, ib, wb, fb, sem):
            pltpu.async_copy(idx_hbm.at[b, ci], ib, sem)
            pltpu.async_copy(wz_hbm.at[b, ci], wb, sem)
            if not is_norm:
                p0 = ci * CHUNK
                pltpu.async_copy(
                    img_hbm.at[b, c0, pl.ds(p0, CHUNK)], fb.at[0], sem)
                pltpu.async_copy(
                    img_hbm.at[b, c0 + 1, pl.ds(p0, CHUNK)], fb.at[1], sem)

        def wait_slot(ib, wb, fb, sem):
            pltpu.make_async_copy(idx_hbm.at[0, 0], ib, sem).wait()
            pltpu.make_async_copy(wz_hbm.at[0, 0], wb, sem).wait()
            if not is_norm:
                pltpu.make_async_copy(
                    img_hbm.at[0, 0, pl.ds(0, CHUNK)], fb.at[0], sem).wait()
                pltpu.make_async_copy(
                    img_hbm.at[0, 0, pl.ds(0, CHUNK)], fb.at[1], sem).wait()

        def compute(ib, wb, fb):
            def substep(o):
                if is_norm:
                    for k in range(4):
                        iv = ib[k, pl.ds(o, LANES)]
                        wv = wb[k, pl.ds(o, LANES)]
                        plsc.addupdate_scatter(acc, [iv], wv)
                else:
                    f0 = fb[0, pl.ds(o, LANES)]
                    f1 = fb[1, pl.ds(o, LANES)]
                    for k in range(4):
                        iv = ib[k, pl.ds(o, LANES)]
                        wv = wb[k, pl.ds(o, LANES)]
                        plsc.addupdate_scatter(acc, [iv], wv * f0)
                        plsc.addupdate_scatter(acc, [iv + HW], wv * f1)

            def step(s, _):
                o = s * (2 * LANES)
                substep(o)
                substep(o + LANES)
                return _

            lax.fori_loop(0, CHUNK // (2 * LANES), step, None)

        fire(0, ib0, wb0, fb0, sem0)
        zero_acc()

        def pair_body(j, _):
            ci = 2 * j
            fire(ci + 1, ib1, wb1, fb1, sem1)
            wait_slot(ib0, wb0, fb0, sem0)
            compute(ib0, wb0, fb0)

            @pl.when(ci + 2 < NCHUNK)
            def _():
                fire(ci + 2, ib0, wb0, fb0, sem0)

            wait_slot(ib1, wb1, fb1, sem1)
            compute(ib1, wb1, fb1)
            return _

        lax.fori_loop(0, NPAIR, pair_body, None)
        pltpu.sync_copy(acc.at[pl.ds(0, HW)], out_hbm.at[b, c0])
        pltpu.sync_copy(acc.at[pl.ds(HW, HW)], out_hbm.at[b, c0 + 1])

    def reg_body(ui, _):
        rid = wid + ui * NW
        process(rid // NGRP, 2 * (rid % NGRP), False)
        return _

    lax.fori_loop(0, REG_ROUNDS, reg_body, None)

    @pl.when(wid >= NW - N)
    def _():
        process(wid - (NW - N), C, True)


@functools.cache
def _get_splat():
    return pl.kernel(
        _splat_body,
        out_type=jax.ShapeDtypeStruct((N, NCH, HW), jnp.float32),
        mesh=plsc.VectorSubcoreMesh(
            core_axis_name="c", subcore_axis_name="s",
            num_cores=2, num_subcores=16),
        scratch_types=[
            pltpu.VMEM((2 * HW,), jnp.float32),
            pltpu.VMEM((4, CHUNK), jnp.int32),
            pltpu.VMEM((4, CHUNK), jnp.float32),
            pltpu.VMEM((2, CHUNK), jnp.float32),
            pltpu.VMEM((4, CHUNK), jnp.int32),
            pltpu.VMEM((4, CHUNK), jnp.float32),
            pltpu.VMEM((2, CHUNK), jnp.float32),
            pltpu.SemaphoreType.DMA,
            pltpu.SemaphoreType.DMA,
        ],
        compiler_params=pltpu.CompilerParams(needs_layout_passes=False),
    )


CB = 16


def _norm_body(acc_ref, nrm_ref, o_ref):
    nv = nrm_ref[0, 0:1, :]
    d = jnp.where(nv == 0.0, jnp.ones_like(nv), nv)
    r = acc_ref[0] / d
    o_ref[0] = r.reshape(CB, H, W)


_norm = pl.pallas_call(
    _norm_body,
    grid=(N, C // CB),
    in_specs=[
        pl.BlockSpec((1, CB, HW), lambda b, c: (b, c, 0)),
        pl.BlockSpec((1, 8, HW), lambda b, c: (b, C // 8, 0)),
    ],
    out_specs=pl.BlockSpec((1, CB, H, W), lambda b, c: (b, c, 0, 0)),
    out_shape=jax.ShapeDtypeStruct((N, C, H, W), jnp.float32),
)


@jax.jit
def kernel(img, flow, z):
    idx, wz = _prep(flow, z)
    idxc = idx.reshape(N, 4, NCHUNK, CHUNK).transpose(0, 2, 1, 3)
    wzc = wz.reshape(N, 4, NCHUNK, CHUNK).transpose(0, 2, 1, 3)
    buf = _get_splat()(idxc, wzc, img.reshape(N, C, HW))
    return _norm(buf, buf)
